# baseline (speedup 1.0000x reference)
import functools

import jax
import jax.numpy as jnp
from jax import lax
from jax.experimental import pallas as pl
from jax.experimental.pallas import tpu as pltpu
from jax.experimental.pallas import tpu_sc as plsc

NC = 2
NS = 16
NW = NC * NS

CHUNK = 128
EMBED = 64


def _gather_body(n_chunks, ids_hbm, table_hbm, out_hbm, idx_v, rows_v, sem):
    cid = lax.axis_index("c")
    sid = lax.axis_index("s")
    wid = sid * NC + cid

    pltpu.sync_copy(ids_hbm.at[wid], idx_v)

    def step(j, _):
        pltpu.async_copy(table_hbm.at[idx_v.at[j]], rows_v, sem).wait()
        pltpu.sync_copy(rows_v, out_hbm.at[wid, j])
        return _

    lax.fori_loop(0, n_chunks, step, None)


def _make_kernel(n_chunks):
    return pl.kernel(
        functools.partial(_gather_body, n_chunks),
        out_type=jax.ShapeDtypeStruct((NW, n_chunks, CHUNK, EMBED), jnp.float32),
        mesh=plsc.VectorSubcoreMesh(core_axis_name="c", subcore_axis_name="s"),
        scratch_types=[
            pltpu.VMEM((n_chunks, CHUNK), jnp.int32),
            pltpu.VMEM((CHUNK, EMBED), jnp.float32),
            pltpu.SemaphoreType.DMA,
        ],
        compiler_params=pltpu.CompilerParams(use_tc_tiling_on_sc=False),
    )


@jax.jit
def kernel(input_ids, table):
    b, s = input_ids.shape
    total = b * s
    n_chunks = total // (NW * CHUNK)
    ids = input_ids.astype(jnp.int32).reshape(NW, n_chunks, CHUNK)
    out = _make_kernel(n_chunks)(ids, table)
    return out.reshape(b, s, EMBED)


# speedup vs baseline: 1.0470x; 1.0470x over previous
import functools

import jax
import jax.numpy as jnp
from jax import lax
from jax.experimental import pallas as pl
from jax.experimental.pallas import tpu as pltpu
from jax.experimental.pallas import tpu_sc as plsc

NC = 2
NS = 16
NW = NC * NS

CHUNK = 128
EMBED = 64


NBUF = 5


def _gather_body(n_chunks, ids_hbm, table_hbm, out_hbm, idx_v, rows_v,
                 gsems, wsems):
    cid = lax.axis_index("c")
    sid = lax.axis_index("s")
    wid = sid * NC + cid
    n_out = n_chunks // NBUF

    pltpu.sync_copy(ids_hbm.at[wid], idx_v)

    def gather(j, b):
        return pltpu.make_async_copy(
            table_hbm.at[idx_v.at[j]], rows_v.at[b], gsems[b])

    def write(j, b):
        return pltpu.make_async_copy(
            rows_v.at[b], out_hbm.at[wid, j], wsems[b])

    for b in range(NBUF):
        gather(b, b).start()

    def step(outer, _):
        for b in range(NBUF):
            j = outer * NBUF + b
            gather(j, b).wait()
            write(j, b).start()
            write(j, b).wait()

            @pl.when(outer < n_out - 1)
            def _():
                gather(j + NBUF, b).start()
        return _

    lax.fori_loop(0, n_out, step, None)


def _make_kernel(n_chunks):
    def body(ids_hbm, table_hbm, out_hbm, *scratch):
        idx_v = scratch[0]
        rows_v = scratch[1]
        gsems = scratch[2:2 + NBUF]
        wsems = scratch[2 + NBUF:]
        _gather_body(n_chunks, ids_hbm, table_hbm, out_hbm, idx_v, rows_v,
                     gsems, wsems)

    return pl.kernel(
        body,
        out_type=jax.ShapeDtypeStruct((NW, n_chunks, CHUNK, EMBED), jnp.float32),
        mesh=plsc.VectorSubcoreMesh(core_axis_name="c", subcore_axis_name="s"),
        scratch_types=[
            pltpu.VMEM((n_chunks, CHUNK), jnp.int32),
            pltpu.VMEM((NBUF, CHUNK, EMBED), jnp.float32),
        ] + [pltpu.SemaphoreType.DMA] * (2 * NBUF),
        compiler_params=pltpu.CompilerParams(use_tc_tiling_on_sc=False),
    )


@jax.jit
def kernel(input_ids, table):
    b, s = input_ids.shape
    total = b * s
    n_chunks = total // (NW * CHUNK)
    ids = input_ids.astype(jnp.int32).reshape(NW, n_chunks, CHUNK)
    out = _make_kernel(n_chunks)(ids, table)
    return out.reshape(b, s, EMBED)
